# initial kernel scaffold (unmeasured)
import jax
import jax.numpy as jnp
from jax import lax
from jax.experimental import pallas as pl
from jax.experimental.pallas import tpu as pltpu

N_DEV = 4
HQ = 8
DH = 128
BLK = 64
N_RES = 4
F32 = jnp.float32
SCALE = 0.08838834764831843

_CompilerParams = getattr(pltpu, "CompilerParams", None) or getattr(
    pltpu, "TPUCompilerParams"
)


def kernel(x, Wq, K_ext, V_ext, Wo):
    _, SQ, D_MODEL = x.shape
    SKV = K_ext.shape[1]
    n_qb = SQ // BLK
    m_per_res = n_qb // N_RES
    rows_per_res = m_per_res * BLK

    def body(x_ref, wq_ref, k_ref, v_ref, wo_ref, out_ref,
             k_loc, v_loc, k_all, v_all, p_all, ctxp,
             k_send_s, k_recv_s, v_send_s, v_recv_s,
             p_send_s, p_recv_s, loc_s):
        my = lax.axis_index("i")

        bsem = pltpu.get_barrier_semaphore()
        for d in range(1, N_DEV):
            peer = lax.rem(my + d, N_DEV)
            pl.semaphore_signal(bsem, inc=1, device_id=(peer,),
                                device_id_type=pl.DeviceIdType.MESH)
        pl.semaphore_wait(bsem, N_DEV - 1)

        kv_descs = []
        for d in range(1, N_DEV):
            peer = lax.rem(my + d, N_DEV)
            for h in range(HQ):
                for src_ref, allbuf, ss, rs in (
                    (k_ref, k_all, k_send_s, k_recv_s),
                    (v_ref, v_all, v_send_s, v_recv_s),
                ):
                    c = pltpu.make_async_remote_copy(
                        src_ref=src_ref.at[0, :, peer * HQ + h, :],
                        dst_ref=allbuf.at[d - 1, h],
                        send_sem=ss.at[d - 1],
                        recv_sem=rs.at[d - 1],
                        device_id=(peer,),
                        device_id_type=pl.DeviceIdType.MESH,
                    )
                    c.start()
                    kv_descs.append(c)

        loc_descs = []
        for h in range(HQ):
            ck = pltpu.make_async_copy(
                k_ref.at[0, :, my * HQ + h, :], k_loc.at[h], loc_s.at[0])
            cv = pltpu.make_async_copy(
                v_ref.at[0, :, my * HQ + h, :], v_loc.at[h], loc_s.at[1])
            ck.start()
            cv.start()
            loc_descs += [ck, cv]

        q = jnp.dot(x_ref[0], wq_ref[...], preferred_element_type=F32)

        for c in loc_descs:
            c.wait()
        for c in kv_descs:
            c.wait_recv()
        for c in kv_descs:
            c.wait_send()

        for h in range(HQ):
            qh = q[:, h * DH:(h + 1) * DH]
            kh = [k_loc[h]] + [k_all[d, h] for d in range(N_DEV - 1)]
            vh = [v_loc[h]] + [v_all[d, h] for d in range(N_DEV - 1)]
            for r in range(N_RES):
                rows = [(m * N_RES + r) * BLK for m in range(m_per_res)]
                qr = jnp.concatenate(
                    [qh[o:o + BLK, :] for o in rows], axis=0)
                kr = jnp.concatenate(
                    [c[o:o + BLK, :] for c in kh for o in rows], axis=0)
                vr = jnp.concatenate(
                    [c[o:o + BLK, :] for c in vh for o in rows], axis=0)
                s = lax.dot_general(
                    qr, kr, (((1,), (1,)), ((), ())),
                    preferred_element_type=F32) * SCALE
                mx = jnp.max(s, axis=1, keepdims=True)
                e = jnp.exp(s - mx)
                p = e / jnp.sum(e, axis=1, keepdims=True)
                ctxp[r, :, h * DH:(h + 1) * DH] = jnp.dot(
                    p, vr, preferred_element_type=F32)

        ctx = ctxp[...].reshape(N_RES, m_per_res, BLK, HQ * DH)
        ctx = ctx.transpose(1, 0, 2, 3).reshape(SQ, HQ * DH)
        out_ref[0] = jnp.dot(ctx, wo_ref[...], preferred_element_type=F32)

        p_descs = []
        for d in range(1, N_DEV):
            peer = lax.rem(my + d, N_DEV)
            c = pltpu.make_async_remote_copy(
                src_ref=out_ref.at[0],
                dst_ref=p_all.at[d - 1],
                send_sem=p_send_s.at[d - 1],
                recv_sem=p_recv_s.at[d - 1],
                device_id=(peer,),
                device_id_type=pl.DeviceIdType.MESH,
            )
            c.start()
            p_descs.append(c)
        for c in p_descs:
            c.wait_recv()
        for c in p_descs:
            c.wait_send()
        out_ref[0] = out_ref[0] + p_all[0] + p_all[1] + p_all[2]

    return pl.pallas_call(
        body,
        out_shape=jax.ShapeDtypeStruct((1, SQ, D_MODEL), F32),
        in_specs=[
            pl.BlockSpec(memory_space=pltpu.VMEM),
            pl.BlockSpec(memory_space=pltpu.VMEM),
            pl.BlockSpec(memory_space=pltpu.ANY),
            pl.BlockSpec(memory_space=pltpu.ANY),
            pl.BlockSpec(memory_space=pltpu.VMEM),
        ],
        out_specs=pl.BlockSpec(memory_space=pltpu.VMEM),
        scratch_shapes=[
            pltpu.VMEM((HQ, SKV, DH), F32),
            pltpu.VMEM((HQ, SKV, DH), F32),
            pltpu.VMEM((N_DEV - 1, HQ, SKV, DH), F32),
            pltpu.VMEM((N_DEV - 1, HQ, SKV, DH), F32),
            pltpu.VMEM((N_DEV - 1, SQ, D_MODEL), F32),
            pltpu.VMEM((N_RES, rows_per_res, HQ * DH), F32),
            pltpu.SemaphoreType.DMA((N_DEV - 1,)),
            pltpu.SemaphoreType.DMA((N_DEV - 1,)),
            pltpu.SemaphoreType.DMA((N_DEV - 1,)),
            pltpu.SemaphoreType.DMA((N_DEV - 1,)),
            pltpu.SemaphoreType.DMA((N_DEV - 1,)),
            pltpu.SemaphoreType.DMA((N_DEV - 1,)),
            pltpu.SemaphoreType.DMA((2,)),
        ],
        compiler_params=_CompilerParams(collective_id=0),
    )(x, Wq, K_ext, V_ext, Wo)


# baseline (device time: 303819 ns/iter reference)
import jax
import jax.numpy as jnp
from jax import lax
from jax.experimental import pallas as pl
from jax.experimental.pallas import tpu as pltpu

N_DEV = 4
HQ = 8
DH = 128
BLK = 64
N_RES = 4
F32 = jnp.float32
SCALE = 0.08838834764831843

_CompilerParams = getattr(pltpu, "CompilerParams", None) or getattr(
    pltpu, "TPUCompilerParams"
)


def kernel(x, Wq, K_ext, V_ext, Wo):
    _, SQ, D_MODEL = x.shape
    SKV = K_ext.shape[1]
    HD = HQ * DH
    n_qb = SQ // BLK
    m_per_res = n_qb // N_RES

    def body(x_ref, wq_ref, k_ref, v_ref, wo_ref, out_ref,
             xv, wqv, qv, wov, outv, k_all, v_all,
             k_send_s, k_recv_s, v_send_s, v_recv_s,
             p_send_s, p_recv_s, loc_s):
        my = lax.axis_index("i")

        cx = pltpu.make_async_copy(x_ref.at[0], xv, loc_s.at[2])
        cwq = pltpu.make_async_copy(wq_ref, wqv, loc_s.at[3])
        cwo = pltpu.make_async_copy(wo_ref, wov, loc_s.at[4])
        cx.start()
        cwq.start()
        cwo.start()

        bsem = pltpu.get_barrier_semaphore()
        for d in range(1, N_DEV):
            peer = lax.rem(my + d, N_DEV)
            pl.semaphore_signal(bsem, inc=1, device_id=(peer,),
                                device_id_type=pl.DeviceIdType.MESH)
        pl.semaphore_wait(bsem, N_DEV - 1)

        kv_descs = []
        for d in range(1, N_DEV):
            peer = lax.rem(my + d, N_DEV)
            for h in range(HQ):
                for src_ref, allbuf, ss, rs in (
                    (k_ref, k_all, k_send_s, k_recv_s),
                    (v_ref, v_all, v_send_s, v_recv_s),
                ):
                    c = pltpu.make_async_remote_copy(
                        src_ref=src_ref.at[0, :, peer * HQ + h, :],
                        dst_ref=allbuf.at[d - 1, :, h * DH:(h + 1) * DH],
                        send_sem=ss.at[d - 1],
                        recv_sem=rs.at[d - 1],
                        device_id=(peer,),
                        device_id_type=pl.DeviceIdType.MESH,
                    )
                    c.start()
                    kv_descs.append(c)

        loc_descs = []
        for h in range(HQ):
            ck = pltpu.make_async_copy(
                k_ref.at[0, :, my * HQ + h, :],
                k_all.at[N_DEV - 1, :, h * DH:(h + 1) * DH], loc_s.at[0])
            cv = pltpu.make_async_copy(
                v_ref.at[0, :, my * HQ + h, :],
                v_all.at[N_DEV - 1, :, h * DH:(h + 1) * DH], loc_s.at[1])
            ck.start()
            cv.start()
            loc_descs += [ck, cv]

        cx.wait()
        cwq.wait()
        qv[...] = jnp.dot(xv[...], wqv[...], preferred_element_type=F32)

        for c in loc_descs:
            c.wait()
        for c in kv_descs:
            c.wait_recv()
        for c in kv_descs:
            c.wait_send()

        cwo.wait()
        for r in range(N_RES):
            rows = [(m * N_RES + r) * BLK for m in range(m_per_res)]
            ctx_cols = []
            for h in range(HQ):
                hc = slice(h * DH, (h + 1) * DH)
                qr = jnp.concatenate(
                    [qv[o:o + BLK, hc] for o in rows], axis=0)
                kr = jnp.concatenate(
                    [k_all[c, o:o + BLK, hc]
                     for c in range(N_DEV) for o in rows], axis=0)
                vr = jnp.concatenate(
                    [v_all[c, o:o + BLK, hc]
                     for c in range(N_DEV) for o in rows], axis=0)
                s = lax.dot_general(
                    qr, kr, (((1,), (1,)), ((), ())),
                    preferred_element_type=F32) * SCALE
                mx = jnp.max(s, axis=1, keepdims=True)
                e = jnp.exp(s - mx)
                p = e / jnp.sum(e, axis=1, keepdims=True)
                ctx_cols.append(
                    jnp.dot(p, vr, preferred_element_type=F32))
            ctx_r = jnp.concatenate(ctx_cols, axis=1)
            out_r = jnp.dot(ctx_r, wov[...], preferred_element_type=F32)
            for m in range(m_per_res):
                o = (m * N_RES + r) * BLK
                outv[o:o + BLK, :] = out_r[m * BLK:(m + 1) * BLK, :]

        for d in range(1, N_DEV):
            peer = lax.rem(my + d, N_DEV)
            pl.semaphore_signal(bsem, inc=1, device_id=(peer,),
                                device_id_type=pl.DeviceIdType.MESH)
        pl.semaphore_wait(bsem, N_DEV - 1)

        p_descs = []
        for d in range(1, N_DEV):
            peer = lax.rem(my + d, N_DEV)
            c = pltpu.make_async_remote_copy(
                src_ref=outv,
                dst_ref=k_all.at[d - 1],
                send_sem=p_send_s.at[d - 1],
                recv_sem=p_recv_s.at[d - 1],
                device_id=(peer,),
                device_id_type=pl.DeviceIdType.MESH,
            )
            c.start()
            p_descs.append(c)
        for c in p_descs:
            c.wait_recv()
        for c in p_descs:
            c.wait_send()
        outv[...] = outv[...] + k_all[0] + k_all[1] + k_all[2]

        cout = pltpu.make_async_copy(outv, out_ref.at[0], loc_s.at[5])
        cout.start()
        cout.wait()

    return pl.pallas_call(
        body,
        out_shape=jax.ShapeDtypeStruct((1, SQ, D_MODEL), F32),
        in_specs=[
            pl.BlockSpec(memory_space=pl.ANY),
            pl.BlockSpec(memory_space=pl.ANY),
            pl.BlockSpec(memory_space=pl.ANY),
            pl.BlockSpec(memory_space=pl.ANY),
            pl.BlockSpec(memory_space=pl.ANY),
        ],
        out_specs=pl.BlockSpec(memory_space=pl.ANY),
        scratch_shapes=[
            pltpu.VMEM((SQ, D_MODEL), F32),
            pltpu.VMEM((D_MODEL, HD), F32),
            pltpu.VMEM((SQ, HD), F32),
            pltpu.VMEM((HD, D_MODEL), F32),
            pltpu.VMEM((SQ, D_MODEL), F32),
            pltpu.VMEM((N_DEV, SKV, HD), F32),
            pltpu.VMEM((N_DEV, SKV, HD), F32),
            pltpu.SemaphoreType.DMA((N_DEV - 1,)),
            pltpu.SemaphoreType.DMA((N_DEV - 1,)),
            pltpu.SemaphoreType.DMA((N_DEV - 1,)),
            pltpu.SemaphoreType.DMA((N_DEV - 1,)),
            pltpu.SemaphoreType.DMA((N_DEV - 1,)),
            pltpu.SemaphoreType.DMA((N_DEV - 1,)),
            pltpu.SemaphoreType.DMA((6,)),
        ],
        compiler_params=_CompilerParams(
            collective_id=0, vmem_limit_bytes=63 * 1024 * 1024),
    )(x, Wq, K_ext, V_ext, Wo)


# device time: 226693 ns/iter; 1.3402x vs baseline; 1.3402x over previous
import jax
import jax.numpy as jnp
from jax import lax
from jax.experimental import pallas as pl
from jax.experimental.pallas import tpu as pltpu

N_DEV = 4
HQ = 8
H_ALL = 32
DH = 128
BLK = 64
N_RES = 4
F32 = jnp.float32
BF16 = jnp.bfloat16
SCALE = 0.08838834764831843
PIECE = 128

_CompilerParams = getattr(pltpu, "CompilerParams", None) or getattr(
    pltpu, "TPUCompilerParams"
)


def kernel(x, Wq, K_ext, V_ext, Wo):
    _, SQ, D_MODEL = x.shape
    SKV = K_ext.shape[1]
    HD = HQ * DH
    n_qb = SQ // BLK
    m_per_res = n_qb // N_RES
    n_piece = SKV // PIECE

    def body(x_ref, wq_ref, k_ref, v_ref, wo_ref, out_ref,
             xv, wv, qv, outv, stage, k_bf, v_bf, k_all, v_all, p_bf,
             k_send_s, k_recv_s, v_send_s, v_recv_s,
             p_send_s, p_recv_s, conv_s, loc_s):
        my = lax.axis_index("i")

        cx = pltpu.make_async_copy(x_ref.at[0], xv, loc_s.at[2])
        cwq = pltpu.make_async_copy(wq_ref, wv, loc_s.at[3])
        cx.start()
        cwq.start()

        bsem = pltpu.get_barrier_semaphore()
        for d in range(1, N_DEV):
            peer = lax.rem(my + d, N_DEV)
            pl.semaphore_signal(bsem, inc=1, device_id=(peer,),
                                device_id_type=pl.DeviceIdType.MESH)
        pl.semaphore_wait(bsem, N_DEV - 1)

        pieces = [(k_ref, k_bf, p) for p in range(n_piece)] + \
                 [(v_ref, v_bf, p) for p in range(n_piece)]

        def piece_copy(i):
            src, _, p = pieces[i]
            c = pltpu.make_async_copy(
                src.at[pl.ds(p * PIECE, PIECE), :],
                stage.at[i % 2], conv_s.at[i % 2])
            c.start()
            return c

        kv_descs = []

        def start_sends(src_bf, allbuf, ss, rs):
            for d in range(1, N_DEV):
                peer = lax.rem(my + d, N_DEV)
                c = pltpu.make_async_remote_copy(
                    src_ref=src_bf.at[:, pl.ds(peer * HD, HD)],
                    dst_ref=allbuf.at[d - 1],
                    send_sem=ss.at[d - 1],
                    recv_sem=rs.at[d - 1],
                    device_id=(peer,),
                    device_id_type=pl.DeviceIdType.MESH,
                )
                c.start()
                kv_descs.append(c)

        inflight = [piece_copy(0), piece_copy(1)]
        for i in range(len(pieces)):
            inflight[i % 2].wait()
            _, dst_bf, p = pieces[i]
            dst_bf[p * PIECE:(p + 1) * PIECE, :] = (
                stage[i % 2].astype(BF16))
            if i + 2 < len(pieces):
                inflight[i % 2] = piece_copy(i + 2)
            if i == n_piece - 1:
                start_sends(k_bf, k_all, k_send_s, k_recv_s)
        start_sends(v_bf, v_all, v_send_s, v_recv_s)

        loc_descs = []
        ck = pltpu.make_async_copy(
            k_bf.at[:, pl.ds(my * HD, HD)], k_all.at[N_DEV - 1], loc_s.at[0])
        cv = pltpu.make_async_copy(
            v_bf.at[:, pl.ds(my * HD, HD)], v_all.at[N_DEV - 1], loc_s.at[1])
        ck.start()
        cv.start()
        loc_descs += [ck, cv]

        cx.wait()
        cwq.wait()
        qv[...] = jnp.dot(xv[...], wv[...],
                          preferred_element_type=F32).astype(BF16)
        cwo = pltpu.make_async_copy(wo_ref, wv, loc_s.at[4])
        cwo.start()

        for c in loc_descs:
            c.wait()
        for c in kv_descs:
            c.wait_recv()
        for c in kv_descs:
            c.wait_send()

        cwo.wait()
        for r in range(N_RES):
            rows = [(m * N_RES + r) * BLK for m in range(m_per_res)]
            ctx_cols = []
            for h in range(HQ):
                hc = slice(h * DH, (h + 1) * DH)
                qr = jnp.concatenate(
                    [qv[o:o + BLK, hc] for o in rows], axis=0)
                kr = jnp.concatenate(
                    [k_all[c, o:o + BLK, hc]
                     for c in range(N_DEV) for o in rows], axis=0)
                vr = jnp.concatenate(
                    [v_all[c, o:o + BLK, hc]
                     for c in range(N_DEV) for o in rows], axis=0)
                s = lax.dot_general(
                    qr, kr, (((1,), (1,)), ((), ())),
                    preferred_element_type=F32) * SCALE
                mx = jnp.max(s, axis=1, keepdims=True)
                e = jnp.exp(s - mx)
                p = (e / jnp.sum(e, axis=1, keepdims=True)).astype(BF16)
                ctx_cols.append(
                    jnp.dot(p, vr, preferred_element_type=F32))
            ctx_r = jnp.concatenate(ctx_cols, axis=1)
            out_r = jnp.dot(ctx_r, wv[...], preferred_element_type=F32)
            for m in range(m_per_res):
                o = (m * N_RES + r) * BLK
                outv[o:o + BLK, :] = out_r[m * BLK:(m + 1) * BLK, :]

        for d in range(1, N_DEV):
            peer = lax.rem(my + d, N_DEV)
            pl.semaphore_signal(bsem, inc=1, device_id=(peer,),
                                device_id_type=pl.DeviceIdType.MESH)
        pl.semaphore_wait(bsem, N_DEV - 1)

        p_bf[...] = outv[...].astype(BF16)
        p_descs = []
        for d in range(1, N_DEV):
            peer = lax.rem(my + d, N_DEV)
            c = pltpu.make_async_remote_copy(
                src_ref=p_bf,
                dst_ref=k_all.at[d - 1],
                send_sem=p_send_s.at[d - 1],
                recv_sem=p_recv_s.at[d - 1],
                device_id=(peer,),
                device_id_type=pl.DeviceIdType.MESH,
            )
            c.start()
            p_descs.append(c)
        for c in p_descs:
            c.wait_recv()
        for c in p_descs:
            c.wait_send()
        outv[...] = (outv[...]
                     + k_all[0].astype(F32)
                     + k_all[1].astype(F32)
                     + k_all[2].astype(F32))

        cout = pltpu.make_async_copy(outv, out_ref.at[0], loc_s.at[5])
        cout.start()
        cout.wait()

    return pl.pallas_call(
        body,
        out_shape=jax.ShapeDtypeStruct((1, SQ, D_MODEL), F32),
        in_specs=[pl.BlockSpec(memory_space=pl.ANY)] * 5,
        out_specs=pl.BlockSpec(memory_space=pl.ANY),
        scratch_shapes=[
            pltpu.VMEM((SQ, D_MODEL), F32),
            pltpu.VMEM((D_MODEL, HD), F32),
            pltpu.VMEM((SQ, HD), BF16),
            pltpu.VMEM((SQ, D_MODEL), F32),
            pltpu.VMEM((2, PIECE, H_ALL * DH), F32),
            pltpu.VMEM((SKV, H_ALL * DH), BF16),
            pltpu.VMEM((SKV, H_ALL * DH), BF16),
            pltpu.VMEM((N_DEV, SKV, HD), BF16),
            pltpu.VMEM((N_DEV, SKV, HD), BF16),
            pltpu.VMEM((SQ, D_MODEL), BF16),
            pltpu.SemaphoreType.DMA((N_DEV - 1,)),
            pltpu.SemaphoreType.DMA((N_DEV - 1,)),
            pltpu.SemaphoreType.DMA((N_DEV - 1,)),
            pltpu.SemaphoreType.DMA((N_DEV - 1,)),
            pltpu.SemaphoreType.DMA((N_DEV - 1,)),
            pltpu.SemaphoreType.DMA((N_DEV - 1,)),
            pltpu.SemaphoreType.DMA((2,)),
            pltpu.SemaphoreType.DMA((6,)),
        ],
        compiler_params=_CompilerParams(
            collective_id=0, vmem_limit_bytes=63 * 1024 * 1024),
    )(x, Wq, K_ext.reshape(SKV, H_ALL * DH), V_ext.reshape(SKV, H_ALL * DH),
      Wo)


# device time: 186031 ns/iter; 1.6332x vs baseline; 1.2186x over previous
import jax
import jax.numpy as jnp
from jax import lax
from jax.experimental import pallas as pl
from jax.experimental.pallas import tpu as pltpu

N_DEV = 4
HQ = 8
H_ALL = 32
DH = 128
BLK = 64
N_RES = 4
F32 = jnp.float32
BF16 = jnp.bfloat16
SCALE = 0.08838834764831843
DEPTH = 4

_CompilerParams = getattr(pltpu, "CompilerParams", None) or getattr(
    pltpu, "TPUCompilerParams"
)


def kernel(x, Wq, K_ext, V_ext, Wo):
    _, SQ, D_MODEL = x.shape
    SKV = K_ext.shape[1]
    HD = HQ * DH
    n_qb = SQ // BLK
    m_per_res = n_qb // N_RES

    def body(x_ref, wq_ref, k_ref, v_ref, wo_ref, out_ref,
             xv, wv, qv, outv, stage, k_bf, v_bf, k_all, v_all,
             p_send, p_recv,
             k_send_s, k_recv_s, v_send_s, v_recv_s,
             p_send_s, p_recv_s, conv_s, loc_s):
        my = lax.axis_index("i")

        cx = pltpu.make_async_copy(x_ref.at[0], xv, loc_s.at[2])
        cwq = pltpu.make_async_copy(wq_ref, wv, loc_s.at[3])
        cx.start()
        cwq.start()

        bsem = pltpu.get_barrier_semaphore()
        for d in range(1, N_DEV):
            peer = lax.rem(my + d, N_DEV)
            pl.semaphore_signal(bsem, inc=1, device_id=(peer,),
                                device_id_type=pl.DeviceIdType.MESH)
        pl.semaphore_wait(bsem, N_DEV - 1)

        pieces = [(k_ref, k_bf, h) for h in range(H_ALL)] + \
                 [(v_ref, v_bf, h) for h in range(H_ALL)]

        def piece_copy(i):
            src, _, h = pieces[i]
            c = pltpu.make_async_copy(
                src.at[0, :, h, :], stage.at[i % DEPTH], conv_s.at[i % DEPTH])
            c.start()
            return c

        kv_descs = []
        loc_descs = []

        def start_sends(src_bf, allbuf, ss, rs, loc_sem):
            for d in range(1, N_DEV):
                peer = lax.rem(my + d, N_DEV)
                for h in range(HQ):
                    c = pltpu.make_async_remote_copy(
                        src_ref=src_bf.at[peer * HQ + h],
                        dst_ref=allbuf.at[d - 1, h],
                        send_sem=ss.at[d - 1],
                        recv_sem=rs.at[d - 1],
                        device_id=(peer,),
                        device_id_type=pl.DeviceIdType.MESH,
                    )
                    c.start()
                    kv_descs.append(c)
            for h in range(HQ):
                c = pltpu.make_async_copy(
                    src_bf.at[my * HQ + h], allbuf.at[N_DEV - 1, h], loc_sem)
                c.start()
                loc_descs.append(c)

        inflight = [piece_copy(i) for i in range(DEPTH)]
        for i in range(len(pieces)):
            inflight[i % DEPTH].wait()
            _, dst_bf, h = pieces[i]
            dst_bf[h] = stage[i % DEPTH].astype(BF16)
            if i + DEPTH < len(pieces):
                inflight[i % DEPTH] = piece_copy(i + DEPTH)
            if i == H_ALL - 1:
                start_sends(k_bf, k_all, k_send_s, k_recv_s, loc_s.at[0])
        start_sends(v_bf, v_all, v_send_s, v_recv_s, loc_s.at[1])

        cx.wait()
        cwq.wait()
        qv[...] = jnp.dot(xv[...], wv[...],
                          preferred_element_type=F32).astype(BF16)
        cwo = pltpu.make_async_copy(wo_ref, wv, loc_s.at[4])
        cwo.start()

        for c in loc_descs:
            c.wait()
        for c in kv_descs:
            c.wait_recv()
        for c in kv_descs:
            c.wait_send()

        cwo.wait()
        for r in range(N_RES):
            rows = [(m * N_RES + r) * BLK for m in range(m_per_res)]
            ctx_cols = []
            for h in range(HQ):
                hc = slice(h * DH, (h + 1) * DH)
                qr = jnp.concatenate(
                    [qv[o:o + BLK, hc] for o in rows], axis=0)
                kr = jnp.concatenate(
                    [k_all[c, h, o:o + BLK, :]
                     for c in range(N_DEV) for o in rows], axis=0)
                vr = jnp.concatenate(
                    [v_all[c, h, o:o + BLK, :]
                     for c in range(N_DEV) for o in rows], axis=0)
                s = lax.dot_general(
                    qr, kr, (((1,), (1,)), ((), ())),
                    preferred_element_type=F32) * SCALE
                mx = jnp.max(s, axis=1, keepdims=True)
                e = jnp.exp(s - mx)
                p = (e / jnp.sum(e, axis=1, keepdims=True)).astype(BF16)
                ctx_cols.append(
                    jnp.dot(p, vr, preferred_element_type=F32))
            ctx_r = jnp.concatenate(ctx_cols, axis=1)
            out_r = jnp.dot(ctx_r, wv[...], preferred_element_type=F32)
            for m in range(m_per_res):
                o = (m * N_RES + r) * BLK
                outv[o:o + BLK, :] = out_r[m * BLK:(m + 1) * BLK, :]

        for d in range(1, N_DEV):
            peer = lax.rem(my + d, N_DEV)
            pl.semaphore_signal(bsem, inc=1, device_id=(peer,),
                                device_id_type=pl.DeviceIdType.MESH)
        pl.semaphore_wait(bsem, N_DEV - 1)

        p_send[...] = outv[...].astype(BF16).reshape(HQ, SQ // HQ, D_MODEL)
        p_descs = []
        for d in range(1, N_DEV):
            peer = lax.rem(my + d, N_DEV)
            c = pltpu.make_async_remote_copy(
                src_ref=p_send,
                dst_ref=p_recv.at[d - 1],
                send_sem=p_send_s.at[d - 1],
                recv_sem=p_recv_s.at[d - 1],
                device_id=(peer,),
                device_id_type=pl.DeviceIdType.MESH,
            )
            c.start()
            p_descs.append(c)
        for c in p_descs:
            c.wait_recv()
        for c in p_descs:
            c.wait_send()
        acc = outv[...]
        for d in range(N_DEV - 1):
            acc = acc + p_recv[d].astype(F32).reshape(SQ, D_MODEL)
        outv[...] = acc

        cout = pltpu.make_async_copy(outv, out_ref.at[0], loc_s.at[5])
        cout.start()
        cout.wait()

    return pl.pallas_call(
        body,
        out_shape=jax.ShapeDtypeStruct((1, SQ, D_MODEL), F32),
        in_specs=[pl.BlockSpec(memory_space=pl.ANY)] * 5,
        out_specs=pl.BlockSpec(memory_space=pl.ANY),
        scratch_shapes=[
            pltpu.VMEM((SQ, D_MODEL), F32),
            pltpu.VMEM((D_MODEL, HD), F32),
            pltpu.VMEM((SQ, HD), BF16),
            pltpu.VMEM((SQ, D_MODEL), F32),
            pltpu.VMEM((DEPTH, SKV, DH), F32),
            pltpu.VMEM((H_ALL, SKV, DH), BF16),
            pltpu.VMEM((H_ALL, SKV, DH), BF16),
            pltpu.VMEM((N_DEV, HQ, SKV, DH), BF16),
            pltpu.VMEM((N_DEV, HQ, SKV, DH), BF16),
            pltpu.VMEM((HQ, SQ // HQ, D_MODEL), BF16),
            pltpu.VMEM((N_DEV - 1, HQ, SQ // HQ, D_MODEL), BF16),
            pltpu.SemaphoreType.DMA((N_DEV - 1,)),
            pltpu.SemaphoreType.DMA((N_DEV - 1,)),
            pltpu.SemaphoreType.DMA((N_DEV - 1,)),
            pltpu.SemaphoreType.DMA((N_DEV - 1,)),
            pltpu.SemaphoreType.DMA((N_DEV - 1,)),
            pltpu.SemaphoreType.DMA((N_DEV - 1,)),
            pltpu.SemaphoreType.DMA((DEPTH,)),
            pltpu.SemaphoreType.DMA((6,)),
        ],
        compiler_params=_CompilerParams(
            collective_id=0, vmem_limit_bytes=63 * 1024 * 1024),
    )(x, Wq, K_ext, V_ext, Wo)


# device time: 167193 ns/iter; 1.8172x vs baseline; 1.1127x over previous
import jax
import jax.numpy as jnp
from jax import lax
from jax.experimental import pallas as pl
from jax.experimental.pallas import tpu as pltpu

N_DEV = 4
HQ = 8
H_ALL = 32
DH = 128
BLK = 64
N_RES = 4
F32 = jnp.float32
BF16 = jnp.bfloat16
SCALE = 0.08838834764831843
DEPTH = 4

_CompilerParams = getattr(pltpu, "CompilerParams", None) or getattr(
    pltpu, "TPUCompilerParams"
)


def kernel(x, Wq, K_ext, V_ext, Wo):
    _, SQ, D_MODEL = x.shape
    SKV = K_ext.shape[1]
    HD = HQ * DH
    n_qb = SQ // BLK
    m_per_res = n_qb // N_RES

    def body(x_ref, wq_ref, k_ref, v_ref, wo_ref, out_ref,
             xv, wv, qv, outv, stage, k_bf, v_bf, k_all, v_all,
             p_send, p_recv,
             k_send_s, k_recv_s, v_send_s, v_recv_s,
             p_send_s, p_recv_s, conv_s, loc_s):
        my = lax.axis_index("i")

        cx = pltpu.make_async_copy(x_ref.at[0], xv, loc_s.at[2])
        cwq = pltpu.make_async_copy(wq_ref, wv, loc_s.at[3])
        cx.start()
        cwq.start()

        bsem = pltpu.get_barrier_semaphore()
        for d in range(1, N_DEV):
            peer = lax.rem(my + d, N_DEV)
            pl.semaphore_signal(bsem, inc=1, device_id=(peer,),
                                device_id_type=pl.DeviceIdType.MESH)
        pl.semaphore_wait(bsem, N_DEV - 1)

        pieces = [(k_ref, k_bf, h) for h in range(H_ALL)] + \
                 [(v_ref, v_bf, h) for h in range(H_ALL)]

        def piece_copy(i):
            src, _, h = pieces[i]
            c = pltpu.make_async_copy(
                src.at[0, :, h, :], stage.at[i % DEPTH], conv_s.at[i % DEPTH])
            c.start()
            return c

        kv_descs = []
        loc_descs = []

        def start_sends(src_bf, allbuf, ss, rs, loc_sem):
            for d in range(1, N_DEV):
                peer = lax.rem(my + d, N_DEV)
                for h in range(HQ):
                    c = pltpu.make_async_remote_copy(
                        src_ref=src_bf.at[peer * HQ + h],
                        dst_ref=allbuf.at[d - 1, h],
                        send_sem=ss.at[d - 1],
                        recv_sem=rs.at[d - 1],
                        device_id=(peer,),
                        device_id_type=pl.DeviceIdType.MESH,
                    )
                    c.start()
                    kv_descs.append(c)
            for h in range(HQ):
                c = pltpu.make_async_copy(
                    src_bf.at[my * HQ + h], allbuf.at[N_DEV - 1, h], loc_sem)
                c.start()
                loc_descs.append(c)

        inflight = [piece_copy(i) for i in range(DEPTH)]
        for i in range(len(pieces)):
            inflight[i % DEPTH].wait()
            _, dst_bf, h = pieces[i]
            dst_bf[h] = stage[i % DEPTH].astype(BF16)
            if i + DEPTH < len(pieces):
                inflight[i % DEPTH] = piece_copy(i + DEPTH)
            if i == H_ALL - 1:
                start_sends(k_bf, k_all, k_send_s, k_recv_s, loc_s.at[0])
        start_sends(v_bf, v_all, v_send_s, v_recv_s, loc_s.at[1])

        cx.wait()
        cwq.wait()
        qv[...] = jnp.dot(xv[...], wv[...],
                          preferred_element_type=F32).astype(BF16)
        cwo = pltpu.make_async_copy(wo_ref, wv, loc_s.at[4])
        cwo.start()

        for c in loc_descs:
            c.wait()
        for c in kv_descs:
            c.wait_recv()
        for c in kv_descs:
            c.wait_send()

        cwo.wait()
        p_descs = []
        for r in range(N_RES):
            rows = [(m * N_RES + r) * BLK for m in range(m_per_res)]
            ctx_cols = []
            for h in range(HQ):
                hc = slice(h * DH, (h + 1) * DH)
                qr = jnp.concatenate(
                    [qv[o:o + BLK, hc] for o in rows], axis=0)
                kr = jnp.concatenate(
                    [k_all[c, h, o:o + BLK, :]
                     for c in range(N_DEV) for o in rows], axis=0)
                vr = jnp.concatenate(
                    [v_all[c, h, o:o + BLK, :]
                     for c in range(N_DEV) for o in rows], axis=0)
                s = lax.dot_general(
                    qr, kr, (((1,), (1,)), ((), ())),
                    preferred_element_type=F32) * SCALE
                mx = jnp.max(s, axis=1, keepdims=True)
                e = jnp.exp(s - mx)
                p = (e / jnp.sum(e, axis=1, keepdims=True)).astype(BF16)
                ctx_cols.append(
                    jnp.dot(p, vr, preferred_element_type=F32))
            ctx_r = jnp.concatenate(ctx_cols, axis=1)
            out_r = jnp.dot(ctx_r, wv[...], preferred_element_type=F32)
            rr = r * m_per_res * BLK
            outv[rr:rr + m_per_res * BLK, :] = out_r
            p_send[r] = out_r.astype(BF16)
            for d in range(1, N_DEV):
                peer = lax.rem(my + d, N_DEV)
                c = pltpu.make_async_remote_copy(
                    src_ref=p_send.at[r],
                    dst_ref=p_recv.at[d - 1, r],
                    send_sem=p_send_s.at[d - 1],
                    recv_sem=p_recv_s.at[d - 1],
                    device_id=(peer,),
                    device_id_type=pl.DeviceIdType.MESH,
                )
                c.start()
                p_descs.append(c)

        for c in p_descs:
            c.wait_recv()
        for c in p_descs:
            c.wait_send()
        acc = outv[...]
        for d in range(N_DEV - 1):
            acc = acc + p_recv[d].astype(F32).reshape(SQ, D_MODEL)
        outv[...] = acc

        out_descs = []
        for r in range(N_RES):
            for m in range(m_per_res):
                c = pltpu.make_async_copy(
                    outv.at[pl.ds((r * m_per_res + m) * BLK, BLK), :],
                    out_ref.at[0, pl.ds((m * N_RES + r) * BLK, BLK), :],
                    loc_s.at[5])
                c.start()
                out_descs.append(c)
        for c in out_descs:
            c.wait()

    return pl.pallas_call(
        body,
        out_shape=jax.ShapeDtypeStruct((1, SQ, D_MODEL), F32),
        in_specs=[pl.BlockSpec(memory_space=pl.ANY)] * 5,
        out_specs=pl.BlockSpec(memory_space=pl.ANY),
        scratch_shapes=[
            pltpu.VMEM((SQ, D_MODEL), F32),
            pltpu.VMEM((D_MODEL, HD), F32),
            pltpu.VMEM((SQ, HD), BF16),
            pltpu.VMEM((SQ, D_MODEL), F32),
            pltpu.VMEM((DEPTH, SKV, DH), F32),
            pltpu.VMEM((H_ALL, SKV, DH), BF16),
            pltpu.VMEM((H_ALL, SKV, DH), BF16),
            pltpu.VMEM((N_DEV, HQ, SKV, DH), BF16),
            pltpu.VMEM((N_DEV, HQ, SKV, DH), BF16),
            pltpu.VMEM((N_RES, SQ // N_RES, D_MODEL), BF16),
            pltpu.VMEM((N_DEV - 1, N_RES, SQ // N_RES, D_MODEL), BF16),
            pltpu.SemaphoreType.DMA((N_DEV - 1,)),
            pltpu.SemaphoreType.DMA((N_DEV - 1,)),
            pltpu.SemaphoreType.DMA((N_DEV - 1,)),
            pltpu.SemaphoreType.DMA((N_DEV - 1,)),
            pltpu.SemaphoreType.DMA((N_DEV - 1,)),
            pltpu.SemaphoreType.DMA((N_DEV - 1,)),
            pltpu.SemaphoreType.DMA((DEPTH,)),
            pltpu.SemaphoreType.DMA((6,)),
        ],
        compiler_params=_CompilerParams(
            collective_id=0, vmem_limit_bytes=63 * 1024 * 1024),
    )(x, Wq, K_ext, V_ext, Wo)
